# baseline (speedup 1.0000x reference)
import functools

import jax
import jax.numpy as jnp
from jax import lax
from jax.experimental import pallas as pl
from jax.experimental.pallas import tpu as pltpu
from jax.experimental.pallas import tpu_sc as plsc

B = 4096
D = 64
NUM_CORES = 2
NUM_SUBCORES = 16
LANES = 16
NW = NUM_CORES * NUM_SUBCORES
BPW = B // NW
GROUPS = BPW // LANES

_mesh = plsc.VectorSubcoreMesh(core_axis_name="c", subcore_axis_name="s")


@functools.partial(
    pl.kernel,
    out_type=jax.ShapeDtypeStruct((B,), jnp.float32),
    mesh=_mesh,
    compiler_params=pltpu.CompilerParams(needs_layout_passes=False, use_tc_tiling_on_sc=False),
    scratch_types=[
        pltpu.VMEM((BPW,), jnp.int32),
        pltpu.VMEM((BPW,), jnp.int32),
        pltpu.VMEM((BPW,), jnp.int32),
        pltpu.VMEM((BPW, D), jnp.float32),
        pltpu.VMEM((BPW, D), jnp.float32),
        pltpu.VMEM((BPW, D), jnp.float32),
        pltpu.VMEM((BPW,), jnp.float32),
        pltpu.SemaphoreType.DMA,
        pltpu.SemaphoreType.DMA,
        pltpu.SemaphoreType.DMA,
    ],
)
def _l1_score_kernel(heads, rels, tails, etab, rtab, out,
                     hidx, ridx, tidx, hrows, rrows, trows, sums,
                     sem_h, sem_r, sem_t):
    wid = lax.axis_index("s") * NUM_CORES + lax.axis_index("c")
    base = wid * BPW

    pltpu.sync_copy(heads.at[pl.ds(base, BPW)], hidx)
    pltpu.sync_copy(rels.at[pl.ds(base, BPW)], ridx)
    pltpu.sync_copy(tails.at[pl.ds(base, BPW)], tidx)

    ch = pltpu.async_copy(etab.at[hidx], hrows, sem_h)
    cr = pltpu.async_copy(rtab.at[ridx], rrows, sem_r)
    ct = pltpu.async_copy(etab.at[tidx], trows, sem_t)
    ch.wait()
    cr.wait()
    ct.wait()

    lanes = lax.iota(jnp.int32, LANES)
    for g in range(GROUPS):
        rows = lanes + g * LANES

        def body(d, acc):
            col = jnp.full((LANES,), d, dtype=jnp.int32)
            h = plsc.load_gather(hrows, [rows, col])
            r = plsc.load_gather(rrows, [rows, col])
            t = plsc.load_gather(trows, [rows, col])
            return acc + jnp.abs(h + r - t)

        acc = lax.fori_loop(0, D, body, jnp.zeros((LANES,), jnp.float32))
        sums[pl.ds(g * LANES, LANES)] = acc

    pltpu.sync_copy(sums, out.at[pl.ds(base, BPW)])


def kernel(sample, entity_embedding, relation_embedding, gamma):
    heads = sample[:, 0]
    rels = sample[:, 1]
    tails = sample[:, 2]
    sums = _l1_score_kernel(heads, rels, tails,
                            entity_embedding, relation_embedding)
    return (gamma - sums)[:, None]


# speedup vs baseline: 2.4251x; 2.4251x over previous
import functools

import jax
import jax.numpy as jnp
from jax import lax
from jax.experimental import pallas as pl
from jax.experimental.pallas import tpu as pltpu
from jax.experimental.pallas import tpu_sc as plsc

B = 4096
D = 64
SUB = 8
NUM_CORES = 2
NUM_SUBCORES = 16
LANES = 16
NW = NUM_CORES * NUM_SUBCORES
BPW = B // NW
GROUPS = BPW // LANES

_mesh = plsc.VectorSubcoreMesh(core_axis_name="c", subcore_axis_name="s")


@functools.partial(
    pl.kernel,
    out_type=jax.ShapeDtypeStruct((B,), jnp.float32),
    mesh=_mesh,
    compiler_params=pltpu.CompilerParams(needs_layout_passes=False),
    scratch_types=[
        pltpu.VMEM((BPW,), jnp.int32),
        pltpu.VMEM((BPW,), jnp.int32),
        pltpu.VMEM((BPW,), jnp.int32),
        pltpu.VMEM((BPW, D), jnp.float32),
        pltpu.VMEM((BPW, D), jnp.float32),
        pltpu.VMEM((BPW, D), jnp.float32),
        pltpu.VMEM((BPW,), jnp.float32),
        pltpu.SemaphoreType.DMA,
        pltpu.SemaphoreType.DMA,
        pltpu.SemaphoreType.DMA,
    ],
)
def _l1_score_kernel(heads, rels, tails, etab, rtab, out,
                     hraw, rraw, traw,
                     hrows, rrows, trows, sums,
                     sem_h, sem_r, sem_t):
    wid = lax.axis_index("s") * NUM_CORES + lax.axis_index("c")
    base = wid * BPW

    pltpu.sync_copy(heads.at[pl.ds(base, BPW)], hraw)
    pltpu.sync_copy(rels.at[pl.ds(base, BPW)], rraw)
    pltpu.sync_copy(tails.at[pl.ds(base, BPW)], traw)

    for g in range(GROUPS):
        sl = pl.ds(g * LANES, LANES)
        hv = hraw[sl]
        rv = rraw[sl]
        tv = traw[sl]
        descs = []
        for j in range(LANES):
            i = g * LANES + j
            descs.append(pltpu.async_copy(
                etab.at[lax.shift_right_logical(hv[j], 3),
                        lax.bitwise_and(hv[j], 7)],
                hrows.at[i], sem_h))
            descs.append(pltpu.async_copy(
                rtab.at[lax.shift_right_logical(rv[j], 3),
                        lax.bitwise_and(rv[j], 7)],
                rrows.at[i], sem_r))
            descs.append(pltpu.async_copy(
                etab.at[lax.shift_right_logical(tv[j], 3),
                        lax.bitwise_and(tv[j], 7)],
                trows.at[i], sem_t))
        for dsc in descs:
            dsc.wait()

    lanes = lax.iota(jnp.int32, LANES)
    for g in range(GROUPS):
        sl = pl.ds(g * LANES, LANES)
        rows = lanes + g * LANES

        def body(d, acc):
            col = jnp.full((LANES,), d, dtype=jnp.int32)
            h = plsc.load_gather(hrows, [rows, col])
            r = plsc.load_gather(rrows, [rows, col])
            t = plsc.load_gather(trows, [rows, col])
            return acc + jnp.abs(h + r - t)

        acc = lax.fori_loop(0, D, body, jnp.zeros((LANES,), jnp.float32))
        sums[sl] = acc

    pltpu.sync_copy(sums, out.at[pl.ds(base, BPW)])


def kernel(sample, entity_embedding, relation_embedding, gamma):
    heads = sample[:, 0]
    rels = sample[:, 1]
    tails = sample[:, 2]
    etab3 = entity_embedding.reshape(-1, SUB, D)
    rtab3 = relation_embedding.reshape(-1, SUB, D)
    sums = _l1_score_kernel(heads, rels, tails, etab3, rtab3)
    return (gamma - sums)[:, None]
